# Initial kernel scaffold; baseline (speedup 1.0000x reference)
import jax
import jax.numpy as jnp
from jax.experimental import pallas as pl


def kernel(type_list, abstract_features, gamma, beta):
    raise NotImplementedError("write your pallas kernel here")


# speedup vs baseline: 5.7703x; 5.7703x over previous
import functools

import jax
import jax.numpy as jnp
from jax.experimental import pallas as pl
from jax.experimental.pallas import tpu as pltpu

_EPS = 1e-5


def _typenorm_body(t_ref, x_ref, g_ref, b_ref, o_ref, *, num_types):
    x = x_ref[...]
    mean = jnp.mean(x, axis=1, keepdims=True)
    xc = x - mean
    var = jnp.mean(xc * xc, axis=1, keepdims=True)
    xhat = xc * jax.lax.rsqrt(var + _EPS)
    t = t_ref[...]
    onehot = (t == jax.lax.broadcasted_iota(
        jnp.int32, (t.shape[0], num_types), 1)).astype(jnp.float32)
    g = jnp.dot(onehot, g_ref[...], preferred_element_type=jnp.float32)
    b = jnp.dot(onehot, b_ref[...], preferred_element_type=jnp.float32)
    o_ref[...] = xhat * g + b


def kernel(type_list, abstract_features, gamma, beta):
    n, d = abstract_features.shape
    num_types = gamma.shape[0]
    t2 = type_list.astype(jnp.int32).reshape(n, 1)

    block_n = 4000
    if n % block_n != 0:
        block_n = 1024
    grid = (pl.cdiv(n, block_n),)

    return pl.pallas_call(
        functools.partial(_typenorm_body, num_types=num_types),
        out_shape=jax.ShapeDtypeStruct((n, d), jnp.float32),
        grid=grid,
        in_specs=[
            pl.BlockSpec((block_n, 1), lambda i: (i, 0)),
            pl.BlockSpec((block_n, d), lambda i: (i, 0)),
            pl.BlockSpec((num_types, d), lambda i: (0, 0)),
            pl.BlockSpec((num_types, d), lambda i: (0, 0)),
        ],
        out_specs=pl.BlockSpec((block_n, d), lambda i: (i, 0)),
        compiler_params=pltpu.CompilerParams(
            dimension_semantics=("parallel",),
        ),
        name="typenorm",
    )(t2, abstract_features, gamma, beta)


# speedup vs baseline: 6.3730x; 1.1045x over previous
import functools

import jax
import jax.numpy as jnp
from jax.experimental import pallas as pl
from jax.experimental.pallas import tpu as pltpu

_EPS = 1e-5


def _typenorm_body(t_ref, x_ref, g_ref, b_ref, o_ref, *, num_types):
    x = x_ref[...]
    mean = jnp.mean(x, axis=1, keepdims=True)
    xc = x - mean
    var = jnp.mean(xc * xc, axis=1, keepdims=True)
    xhat = xc * jax.lax.rsqrt(var + _EPS)
    t = t_ref[...]
    onehot = (t == jax.lax.broadcasted_iota(
        jnp.int32, (t.shape[0], num_types), 1)).astype(jnp.float32)
    g = jnp.dot(onehot, g_ref[...], preferred_element_type=jnp.float32)
    b = jnp.dot(onehot, b_ref[...], preferred_element_type=jnp.float32)
    o_ref[...] = xhat * g + b


def kernel(type_list, abstract_features, gamma, beta):
    n, d = abstract_features.shape
    num_types = gamma.shape[0]
    t2 = type_list.astype(jnp.int32).reshape(n, 1)

    block_n = 8000
    if n % block_n != 0:
        block_n = 1024
    grid = (pl.cdiv(n, block_n),)

    return pl.pallas_call(
        functools.partial(_typenorm_body, num_types=num_types),
        out_shape=jax.ShapeDtypeStruct((n, d), jnp.float32),
        grid=grid,
        in_specs=[
            pl.BlockSpec((block_n, 1), lambda i: (i, 0)),
            pl.BlockSpec((block_n, d), lambda i: (i, 0)),
            pl.BlockSpec((num_types, d), lambda i: (0, 0)),
            pl.BlockSpec((num_types, d), lambda i: (0, 0)),
        ],
        out_specs=pl.BlockSpec((block_n, d), lambda i: (i, 0)),
        compiler_params=pltpu.CompilerParams(
            dimension_semantics=("parallel",),
        ),
        name="typenorm",
    )(t2, abstract_features, gamma, beta)


# speedup vs baseline: 6.5181x; 1.0228x over previous
import functools

import jax
import jax.numpy as jnp
from jax.experimental import pallas as pl
from jax.experimental.pallas import tpu as pltpu

_EPS = 1e-5


def _typenorm_body(t_ref, x_ref, g_ref, b_ref, o_ref, *, num_types):
    x = x_ref[...]
    mean = jnp.mean(x, axis=1, keepdims=True)
    xc = x - mean
    var = jnp.mean(xc * xc, axis=1, keepdims=True)
    xhat = xc * jax.lax.rsqrt(var + _EPS)
    t = t_ref[...]
    onehot = (t == jax.lax.broadcasted_iota(
        jnp.int32, (t.shape[0], num_types), 1)).astype(jnp.float32)
    g = jnp.dot(onehot, g_ref[...], preferred_element_type=jnp.float32)
    b = jnp.dot(onehot, b_ref[...], preferred_element_type=jnp.float32)
    o_ref[...] = xhat * g + b


def kernel(type_list, abstract_features, gamma, beta):
    n, d = abstract_features.shape
    num_types = gamma.shape[0]
    t2 = type_list.astype(jnp.int32).reshape(n, 1)

    block_n = 10000
    if n % block_n != 0:
        block_n = 1024
    grid = (pl.cdiv(n, block_n),)

    return pl.pallas_call(
        functools.partial(_typenorm_body, num_types=num_types),
        out_shape=jax.ShapeDtypeStruct((n, d), jnp.float32),
        grid=grid,
        in_specs=[
            pl.BlockSpec((block_n, 1), lambda i: (i, 0)),
            pl.BlockSpec((block_n, d), lambda i: (i, 0)),
            pl.BlockSpec((num_types, d), lambda i: (0, 0)),
            pl.BlockSpec((num_types, d), lambda i: (0, 0)),
        ],
        out_specs=pl.BlockSpec((block_n, d), lambda i: (i, 0)),
        compiler_params=pltpu.CompilerParams(
            dimension_semantics=("parallel",),
        ),
        name="typenorm",
    )(t2, abstract_features, gamma, beta)
